# baseline (speedup 1.0000x reference)
import functools

import jax
import jax.numpy as jnp
from jax import lax
from jax.experimental import pallas as pl
from jax.experimental.pallas import tpu as pltpu

N_NODES = 10000
N_PAD = 10240
N_EDGES = 160000
HEADS = 8
F_HID = 256
SLOPE = 0.2


def _mm_bias_body(a_ref, b_ref, bias_ref, o_ref):
    o_ref[...] = jnp.dot(a_ref[...], b_ref[...],
                         preferred_element_type=jnp.float32) + bias_ref[...]


def _mm_bias(a, b, bias, bm=512, bn=512):
    m, k = a.shape
    k2, n = b.shape
    bm, bn = min(bm, m), min(bn, n)
    assert k == k2 and m % bm == 0 and n % bn == 0
    return pl.pallas_call(
        _mm_bias_body,
        grid=(m // bm, n // bn),
        in_specs=[
            pl.BlockSpec((bm, k), lambda i, j: (i, 0)),
            pl.BlockSpec((k, bn), lambda i, j: (0, j)),
            pl.BlockSpec((1, bn), lambda i, j: (0, j)),
        ],
        out_specs=pl.BlockSpec((bm, bn), lambda i, j: (i, j)),
        out_shape=jax.ShapeDtypeStruct((m, n), jnp.float32),
    )(a, b, bias.reshape(1, n))


def _mm_nt_body(a_ref, b_ref, o_ref):
    o_ref[...] = lax.dot_general(a_ref[...], b_ref[...],
                                 (((1,), (1,)), ((), ())),
                                 preferred_element_type=jnp.float32)


def _mm_nt(a, b, bm=512, bn=512):
    m, k = a.shape
    n, k2 = b.shape
    bm, bn = min(bm, m), min(bn, n)
    assert k == k2 and m % bm == 0 and n % bn == 0
    return pl.pallas_call(
        _mm_nt_body,
        grid=(m // bm, n // bn),
        in_specs=[
            pl.BlockSpec((bm, k), lambda i, j: (i, 0)),
            pl.BlockSpec((bn, k), lambda i, j: (j, 0)),
        ],
        out_specs=pl.BlockSpec((bm, bn), lambda i, j: (i, j)),
        out_shape=jax.ShapeDtypeStruct((m, n), jnp.float32),
    )(a, b)


def _gat_layer(x_pad, W, b, attn, Wr, src, dst):
    xl = _mm_bias(x_pad, W, b)
    xr = _mm_bias(x_pad, Wr, jnp.zeros((F_HID,), jnp.float32))

    xs = xl[src]
    xd = xl[dst]
    m = jax.nn.leaky_relu(xs + xd, negative_slope=SLOPE).reshape(N_EDGES, HEADS, F_HID)
    alpha = (m * attn[None, :, :]).sum(axis=-1)
    amax = jax.ops.segment_max(alpha, dst, num_segments=N_PAD)
    amax = jnp.where(jnp.isfinite(amax), amax, 0.0)
    w = jnp.exp(alpha - amax[dst])
    denom = jax.ops.segment_sum(w, dst, num_segments=N_PAD)
    msg = xs.reshape(N_EDGES, HEADS, F_HID) * w[..., None]
    acc = jax.ops.segment_sum(msg, dst, num_segments=N_PAD)

    out = acc / (denom[..., None] + 1e-16) + xr[:, None, :]
    out = jax.nn.elu(out)
    return out.mean(axis=1)


def kernel(x, edge_idx, W1, b1, attn1, Wr1, W2, b2, attn2, Wr2):
    x_pad = jnp.zeros((N_PAD, x.shape[1]), jnp.float32).at[:N_NODES].set(x)
    src = edge_idx[0].astype(jnp.int32)
    dst = edge_idx[1].astype(jnp.int32)
    h = _gat_layer(x_pad, W1, b1, attn1, Wr1, src, dst)
    h = _gat_layer(h, W2, b2, attn2, Wr2, src, dst)
    rna = h[:6000]
    dis = h[6000:N_NODES]
    rna_p = jnp.zeros((6144, F_HID), jnp.float32).at[:6000].set(rna)
    dis_p = jnp.zeros((4096, F_HID), jnp.float32).at[:4000].set(dis)
    out = _mm_nt(rna_p, dis_p)
    return out[:6000, :4000]


# speedup vs baseline: 2.9741x; 2.9741x over previous
import functools

import jax
import jax.numpy as jnp
from jax import lax
from jax.experimental import pallas as pl
from jax.experimental.pallas import tpu as pltpu
from jax.experimental.pallas import tpu_sc as plsc

N_NODES = 10000
N_PAD = 10240
N_EDGES = 160000
E_PAD = 160256
HEADS = 8
F_HID = 256
D = HEADS * F_HID
SLOPE = 0.2

NC, NS = 2, 16
NW = NC * NS


def _mesh():
    return plsc.VectorSubcoreMesh(core_axis_name="c", subcore_axis_name="s",
                                  num_cores=NC, num_subcores=NS)


def _mm_bias_body(a_ref, b_ref, bias_ref, o_ref):
    o_ref[...] = jnp.dot(a_ref[...], b_ref[...],
                         preferred_element_type=jnp.float32) + bias_ref[...]


def _mm_bias(a, b, bias, bm=512, bn=512):
    m, k = a.shape
    k2, n = b.shape
    bm, bn = min(bm, m), min(bn, n)
    assert k == k2 and m % bm == 0 and n % bn == 0
    return pl.pallas_call(
        _mm_bias_body,
        grid=(m // bm, n // bn),
        in_specs=[
            pl.BlockSpec((bm, k), lambda i, j: (i, 0)),
            pl.BlockSpec((k, bn), lambda i, j: (0, j)),
            pl.BlockSpec((1, bn), lambda i, j: (0, j)),
        ],
        out_specs=pl.BlockSpec((bm, bn), lambda i, j: (i, j)),
        out_shape=jax.ShapeDtypeStruct((m, n), jnp.float32),
    )(a, b, bias.reshape(1, n))


def _mm_nt_body(a_ref, b_ref, o_ref):
    o_ref[...] = lax.dot_general(a_ref[...], b_ref[...],
                                 (((1,), (1,)), ((), ())),
                                 preferred_element_type=jnp.float32)


def _mm_nt(a, b, bm=512, bn=512):
    m, k = a.shape
    n, k2 = b.shape
    bm, bn = min(bm, m), min(bn, n)
    assert k == k2 and m % bm == 0 and n % bn == 0
    return pl.pallas_call(
        _mm_nt_body,
        grid=(m // bm, n // bn),
        in_specs=[
            pl.BlockSpec((bm, k), lambda i, j: (i, 0)),
            pl.BlockSpec((bn, k), lambda i, j: (j, 0)),
        ],
        out_specs=pl.BlockSpec((bm, bn), lambda i, j: (i, j)),
        out_shape=jax.ShapeDtypeStruct((m, n), jnp.float32),
    )(a, b)


def _sc_gather(table, idx):
    rows_total, d = table.shape
    m = idx.shape[0]
    assert m % NW == 0
    per_w = m // NW
    ch = 32
    assert per_w % ch == 0

    def body(table_hbm, idx_hbm, out_hbm, idx_v, rows_v, sem):
        wid = lax.axis_index("s") * NC + lax.axis_index("c")
        base = wid * per_w

        def step(i, _):
            off = base + i * ch
            pltpu.sync_copy(idx_hbm.at[pl.ds(off, ch)], idx_v)
            pltpu.async_copy(table_hbm.at[idx_v], rows_v, sem).wait()
            pltpu.sync_copy(rows_v, out_hbm.at[pl.ds(off, ch)])
            return 0

        lax.fori_loop(0, per_w // ch, step, 0)

    return pl.kernel(
        body,
        out_type=jax.ShapeDtypeStruct((m, d), jnp.float32),
        mesh=_mesh(),
        scratch_types=[
            pltpu.VMEM((ch,), jnp.int32),
            pltpu.VMEM((ch, d), jnp.float32),
            pltpu.SemaphoreType.DMA,
        ],
    )(table, idx)


def _alpha_body(xs_ref, xd_ref, a_ref, o_ref):
    m = xs_ref[...] + xd_ref[...]
    m = jnp.where(m > 0, m, SLOPE * m)
    o_ref[...] = jnp.dot(m, a_ref[...], preferred_element_type=jnp.float32)


def _alpha(xcat, a_blk, be=512):
    nblk = E_PAD // be
    return pl.pallas_call(
        _alpha_body,
        grid=(nblk,),
        in_specs=[
            pl.BlockSpec((be, D), lambda i: (i, 0)),
            pl.BlockSpec((be, D), lambda i: (i + nblk, 0)),
            pl.BlockSpec((D, 128), lambda i: (0, 0)),
        ],
        out_specs=pl.BlockSpec((be, 128), lambda i: (i, 0)),
        out_shape=jax.ShapeDtypeStruct((E_PAD, 128), jnp.float32),
    )(xcat, xcat, a_blk)


def _gmax_body(a_ref, o_ref):
    @pl.when(pl.program_id(0) == 0)
    def _():
        o_ref[...] = jnp.full_like(o_ref, -1e30)
    o_ref[...] = jnp.maximum(o_ref[...], jnp.max(a_ref[...], axis=0,
                                                 keepdims=True))


def _gmax(alpha, be=2048):
    return pl.pallas_call(
        _gmax_body,
        grid=(E_PAD // be,),
        in_specs=[pl.BlockSpec((be, 128), lambda i: (i, 0))],
        out_specs=pl.BlockSpec((1, 128), lambda i: (0, 0)),
        out_shape=jax.ShapeDtypeStruct((1, 128), jnp.float32),
    )(alpha)


def _msg_body(alpha_ref, gmax_ref, xs_ref, p_ref, msg_ref, wcol_ref, *, be):
    i = pl.program_id(0)
    al = alpha_ref[...]
    w = jnp.exp(al - gmax_ref[...])
    eid = i * be + lax.broadcasted_iota(jnp.int32, (be, 128), 0)
    hid = lax.broadcasted_iota(jnp.int32, (be, 128), 1)
    w = jnp.where((eid < N_EDGES) & (hid < HEADS), w, 0.0)
    wcol_ref[...] = w
    wex = jnp.dot(w, p_ref[...], preferred_element_type=jnp.float32)
    msg_ref[...] = xs_ref[...] * wex


def _msg(xcat, alpha, gmax, p_mat, be=512):
    nblk = E_PAD // be
    return pl.pallas_call(
        functools.partial(_msg_body, be=be),
        grid=(nblk,),
        in_specs=[
            pl.BlockSpec((be, 128), lambda i: (i, 0)),
            pl.BlockSpec((1, 128), lambda i: (0, 0)),
            pl.BlockSpec((be, D), lambda i: (i, 0)),
            pl.BlockSpec((128, D), lambda i: (0, 0)),
        ],
        out_specs=[
            pl.BlockSpec((be, D), lambda i: (i, 0)),
            pl.BlockSpec((be, 128), lambda i: (i, 0)),
        ],
        out_shape=[
            jax.ShapeDtypeStruct((E_PAD, D), jnp.float32),
            jax.ShapeDtypeStruct((E_PAD, 128), jnp.float32),
        ],
    )(alpha, gmax, xcat, p_mat)


def _outstage_body(accf_ref, accw_ref, xr_ref, p_ref, o_ref):
    denom = jnp.maximum(jnp.dot(accw_ref[...], p_ref[...],
                                preferred_element_type=jnp.float32), 1e-30)
    v = accf_ref[...] / denom
    xr = xr_ref[...]
    acc = jnp.zeros_like(xr)
    for h in range(HEADS):
        t = v[:, h * F_HID:(h + 1) * F_HID] + xr
        acc = acc + jnp.where(t > 0, t, jnp.exp(jnp.minimum(t, 0.0)) - 1.0)
    o_ref[...] = acc * (1.0 / HEADS)


def _outstage(accf, accw, xr, p_mat, bn=1024):
    return pl.pallas_call(
        _outstage_body,
        grid=(N_PAD // bn,),
        in_specs=[
            pl.BlockSpec((bn, D), lambda i: (i, 0)),
            pl.BlockSpec((bn, 128), lambda i: (i, 0)),
            pl.BlockSpec((bn, F_HID), lambda i: (i, 0)),
            pl.BlockSpec((128, D), lambda i: (0, 0)),
        ],
        out_specs=pl.BlockSpec((bn, F_HID), lambda i: (i, 0)),
        out_shape=jax.ShapeDtypeStruct((N_PAD, F_HID), jnp.float32),
    )(accf, accw, xr, p_mat)


def _gat_layer(x_pad, W, b, attn, Wr, idx_cat, dst_pad, p_mat):
    xl = _mm_bias(x_pad, W, b)
    xr = _mm_bias(x_pad, Wr, jnp.zeros((F_HID,), jnp.float32))

    xcat = _sc_gather(xl, idx_cat)

    a_blk = jnp.zeros((D, 128), jnp.float32)
    a_blk = a_blk.at[jnp.arange(D), jnp.arange(D) // F_HID].set(attn.reshape(-1))
    alpha = _alpha(xcat, a_blk)
    gmax = _gmax(alpha)
    msgf, wcol = _msg(xcat, alpha, gmax, p_mat)

    accf = jax.ops.segment_sum(msgf, dst_pad, num_segments=N_PAD)
    accw = jax.ops.segment_sum(wcol, dst_pad, num_segments=N_PAD)

    return _outstage(accf, accw, xr, p_mat)


def kernel(x, edge_idx, W1, b1, attn1, Wr1, W2, b2, attn2, Wr2):
    x_pad = jnp.zeros((N_PAD, x.shape[1]), jnp.float32).at[:N_NODES].set(x)
    src = edge_idx[0].astype(jnp.int32)
    dst = edge_idx[1].astype(jnp.int32)
    zpad = jnp.zeros((E_PAD - N_EDGES,), jnp.int32)
    src_pad = jnp.concatenate([src, zpad])
    dst_pad = jnp.concatenate([dst, zpad])
    idx_cat = jnp.concatenate([src_pad, dst_pad])

    p_mat = jnp.zeros((128, D), jnp.float32)
    p_mat = p_mat.at[jnp.arange(D) // F_HID, jnp.arange(D)].set(1.0)

    h = _gat_layer(x_pad, W1, b1, attn1, Wr1, idx_cat, dst_pad, p_mat)
    h = _gat_layer(h, W2, b2, attn2, Wr2, idx_cat, dst_pad, p_mat)

    rna_p = jnp.zeros((6144, F_HID), jnp.float32).at[:6000].set(h[:6000])
    dis_p = jnp.zeros((4096, F_HID), jnp.float32).at[:4000].set(h[6000:N_NODES])
    out = _mm_nt(rna_p, dis_p)
    return out[:6000, :4000]


# speedup vs baseline: 3.1032x; 1.0434x over previous
import functools

import jax
import jax.numpy as jnp
from jax import lax
from jax.experimental import pallas as pl
from jax.experimental.pallas import tpu as pltpu
from jax.experimental.pallas import tpu_sc as plsc

N_NODES = 10000
N_PAD = 10240
N_EDGES = 160000
E_PAD = 160256
HEADS = 8
F_HID = 256
D = HEADS * F_HID
SLOPE = 0.2

NC, NS = 2, 16
NW = NC * NS


def _mesh():
    return plsc.VectorSubcoreMesh(core_axis_name="c", subcore_axis_name="s",
                                  num_cores=NC, num_subcores=NS)


def _mm_bias_body(a_ref, b_ref, bias_ref, o_ref):
    o_ref[...] = jnp.dot(a_ref[...], b_ref[...],
                         preferred_element_type=jnp.float32) + bias_ref[...]


def _mm_bias(a, b, bias, bm=512, bn=512):
    m, k = a.shape
    k2, n = b.shape
    bm, bn = min(bm, m), min(bn, n)
    assert k == k2 and m % bm == 0 and n % bn == 0
    return pl.pallas_call(
        _mm_bias_body,
        grid=(m // bm, n // bn),
        in_specs=[
            pl.BlockSpec((bm, k), lambda i, j: (i, 0)),
            pl.BlockSpec((k, bn), lambda i, j: (0, j)),
            pl.BlockSpec((1, bn), lambda i, j: (0, j)),
        ],
        out_specs=pl.BlockSpec((bm, bn), lambda i, j: (i, j)),
        out_shape=jax.ShapeDtypeStruct((m, n), jnp.float32),
    )(a, b, bias.reshape(1, n))


def _mm_nt_body(a_ref, b_ref, o_ref):
    o_ref[...] = lax.dot_general(a_ref[...], b_ref[...],
                                 (((1,), (1,)), ((), ())),
                                 preferred_element_type=jnp.float32)


def _mm_nt(a, b, bm=512, bn=512):
    m, k = a.shape
    n, k2 = b.shape
    bm, bn = min(bm, m), min(bn, n)
    assert k == k2 and m % bm == 0 and n % bn == 0
    return pl.pallas_call(
        _mm_nt_body,
        grid=(m // bm, n // bn),
        in_specs=[
            pl.BlockSpec((bm, k), lambda i, j: (i, 0)),
            pl.BlockSpec((bn, k), lambda i, j: (j, 0)),
        ],
        out_specs=pl.BlockSpec((bm, bn), lambda i, j: (i, j)),
        out_shape=jax.ShapeDtypeStruct((m, n), jnp.float32),
    )(a, b)


def _sc_gather(table, idx):
    rows_total, d = table.shape
    m = idx.shape[0]
    assert m % NW == 0
    per_w = m // NW
    ch = 16
    assert per_w % ch == 0

    n = per_w // ch

    def body(table_hbm, idx_hbm, out_hbm, idx0, idx1, rows0, rows1,
             sem0, sem1):
        wid = lax.axis_index("s") * NC + lax.axis_index("c")
        base = wid * per_w

        def start(i, idx_v, rows_v, sem):
            pltpu.sync_copy(idx_hbm.at[pl.ds(base + i * ch, ch)], idx_v)
            pltpu.async_copy(table_hbm.at[idx_v], rows_v, sem)

        def finish(i, idx_v, rows_v, sem):
            pltpu.make_async_copy(table_hbm.at[idx_v], rows_v, sem).wait()
            pltpu.sync_copy(rows_v, out_hbm.at[pl.ds(base + i * ch, ch)])

        start(0, idx0, rows0, sem0)

        def step(i, _):
            @pl.when(i % 2 == 0)
            def _():
                @pl.when(i + 1 < n)
                def _():
                    start(i + 1, idx1, rows1, sem1)
                finish(i, idx0, rows0, sem0)

            @pl.when(i % 2 == 1)
            def _():
                @pl.when(i + 1 < n)
                def _():
                    start(i + 1, idx0, rows0, sem0)
                finish(i, idx1, rows1, sem1)
            return 0

        lax.fori_loop(0, n, step, 0)

    return pl.kernel(
        body,
        out_type=jax.ShapeDtypeStruct((m, d), jnp.float32),
        mesh=_mesh(),
        scratch_types=[
            pltpu.VMEM((ch,), jnp.int32),
            pltpu.VMEM((ch,), jnp.int32),
            pltpu.VMEM((ch, d), jnp.float32),
            pltpu.VMEM((ch, d), jnp.float32),
            pltpu.SemaphoreType.DMA,
            pltpu.SemaphoreType.DMA,
        ],
    )(table, idx)


def _alpha_body(xs_ref, xd_ref, a_ref, o_ref):
    m = xs_ref[...] + xd_ref[...]
    m = jnp.where(m > 0, m, SLOPE * m)
    o_ref[...] = jnp.dot(m, a_ref[...], preferred_element_type=jnp.float32)


def _alpha(xcat, a_blk, be=512):
    nblk = E_PAD // be
    return pl.pallas_call(
        _alpha_body,
        grid=(nblk,),
        in_specs=[
            pl.BlockSpec((be, D), lambda i: (i, 0)),
            pl.BlockSpec((be, D), lambda i: (i + nblk, 0)),
            pl.BlockSpec((D, 128), lambda i: (0, 0)),
        ],
        out_specs=pl.BlockSpec((be, 128), lambda i: (i, 0)),
        out_shape=jax.ShapeDtypeStruct((E_PAD, 128), jnp.float32),
    )(xcat, xcat, a_blk)


def _gmax_body(a_ref, o_ref):
    @pl.when(pl.program_id(0) == 0)
    def _():
        o_ref[...] = jnp.full_like(o_ref, -1e30)
    o_ref[...] = jnp.maximum(o_ref[...], jnp.max(a_ref[...], axis=0,
                                                 keepdims=True))


def _gmax(alpha, be=2048):
    return pl.pallas_call(
        _gmax_body,
        grid=(E_PAD // be,),
        in_specs=[pl.BlockSpec((be, 128), lambda i: (i, 0))],
        out_specs=pl.BlockSpec((1, 128), lambda i: (0, 0)),
        out_shape=jax.ShapeDtypeStruct((1, 128), jnp.float32),
    )(alpha)


def _msg_body(alpha_ref, gmax_ref, xs_ref, p_ref, msg_ref, wcol_ref, *, be):
    i = pl.program_id(0)
    al = alpha_ref[...]
    w = jnp.exp(al - gmax_ref[...])
    eid = i * be + lax.broadcasted_iota(jnp.int32, (be, 128), 0)
    hid = lax.broadcasted_iota(jnp.int32, (be, 128), 1)
    w = jnp.where((eid < N_EDGES) & (hid < HEADS), w, 0.0)
    wcol_ref[...] = w
    wex = jnp.dot(w, p_ref[...], preferred_element_type=jnp.float32)
    msg_ref[...] = xs_ref[...] * wex


def _msg(xcat, alpha, gmax, p_mat, be=512):
    nblk = E_PAD // be
    return pl.pallas_call(
        functools.partial(_msg_body, be=be),
        grid=(nblk,),
        in_specs=[
            pl.BlockSpec((be, 128), lambda i: (i, 0)),
            pl.BlockSpec((1, 128), lambda i: (0, 0)),
            pl.BlockSpec((be, D), lambda i: (i, 0)),
            pl.BlockSpec((128, D), lambda i: (0, 0)),
        ],
        out_specs=[
            pl.BlockSpec((be, D), lambda i: (i, 0)),
            pl.BlockSpec((be, 128), lambda i: (i, 0)),
        ],
        out_shape=[
            jax.ShapeDtypeStruct((E_PAD, D), jnp.float32),
            jax.ShapeDtypeStruct((E_PAD, 128), jnp.float32),
        ],
    )(alpha, gmax, xcat, p_mat)


def _outstage_body(accf_ref, accw_ref, xr_ref, p_ref, o_ref):
    denom = jnp.maximum(jnp.dot(accw_ref[...], p_ref[...],
                                preferred_element_type=jnp.float32), 1e-30)
    v = accf_ref[...] / denom
    xr = xr_ref[...]
    acc = jnp.zeros_like(xr)
    for h in range(HEADS):
        t = v[:, h * F_HID:(h + 1) * F_HID] + xr
        acc = acc + jnp.where(t > 0, t, jnp.exp(jnp.minimum(t, 0.0)) - 1.0)
    o_ref[...] = acc * (1.0 / HEADS)


def _outstage(accf, accw, xr, p_mat, bn=1024):
    return pl.pallas_call(
        _outstage_body,
        grid=(N_PAD // bn,),
        in_specs=[
            pl.BlockSpec((bn, D), lambda i: (i, 0)),
            pl.BlockSpec((bn, 128), lambda i: (i, 0)),
            pl.BlockSpec((bn, F_HID), lambda i: (i, 0)),
            pl.BlockSpec((128, D), lambda i: (0, 0)),
        ],
        out_specs=pl.BlockSpec((bn, F_HID), lambda i: (i, 0)),
        out_shape=jax.ShapeDtypeStruct((N_PAD, F_HID), jnp.float32),
    )(accf, accw, xr, p_mat)


def _gat_layer(x_pad, W, b, attn, Wr, idx_cat, dst_pad, p_mat):
    xl = _mm_bias(x_pad, W, b)
    xr = _mm_bias(x_pad, Wr, jnp.zeros((F_HID,), jnp.float32))

    xcat = _sc_gather(xl, idx_cat)

    a_blk = jnp.zeros((D, 128), jnp.float32)
    a_blk = a_blk.at[jnp.arange(D), jnp.arange(D) // F_HID].set(attn.reshape(-1))
    alpha = _alpha(xcat, a_blk)
    gmax = _gmax(alpha)
    msgf, wcol = _msg(xcat, alpha, gmax, p_mat)

    accf = jax.ops.segment_sum(msgf, dst_pad, num_segments=N_PAD)
    accw = jax.ops.segment_sum(wcol, dst_pad, num_segments=N_PAD)

    return _outstage(accf, accw, xr, p_mat)


def kernel(x, edge_idx, W1, b1, attn1, Wr1, W2, b2, attn2, Wr2):
    x_pad = jnp.zeros((N_PAD, x.shape[1]), jnp.float32).at[:N_NODES].set(x)
    src = edge_idx[0].astype(jnp.int32)
    dst = edge_idx[1].astype(jnp.int32)
    zpad = jnp.zeros((E_PAD - N_EDGES,), jnp.int32)
    src_pad = jnp.concatenate([src, zpad])
    dst_pad = jnp.concatenate([dst, zpad])
    idx_cat = jnp.concatenate([src_pad, dst_pad])

    p_mat = jnp.zeros((128, D), jnp.float32)
    p_mat = p_mat.at[jnp.arange(D) // F_HID, jnp.arange(D)].set(1.0)

    h = _gat_layer(x_pad, W1, b1, attn1, Wr1, idx_cat, dst_pad, p_mat)
    h = _gat_layer(h, W2, b2, attn2, Wr2, idx_cat, dst_pad, p_mat)

    rna_p = jnp.zeros((6144, F_HID), jnp.float32).at[:6000].set(h[:6000])
    dis_p = jnp.zeros((4096, F_HID), jnp.float32).at[:4000].set(h[6000:N_NODES])
    out = _mm_nt(rna_p, dis_p)
    return out[:6000, :4000]
